# baseline (speedup 1.0000x reference)
import functools

import jax
import jax.numpy as jnp
from jax import lax
from jax.experimental import pallas as pl
from jax.experimental.pallas import tpu as pltpu
from jax.experimental.pallas import tpu_sc as plsc

N = 50000
K = 16
E = 1600000

NC = 2
NS = 16
NW = NC * NS
L = 16
B = 400

PER_W = E // NW
CHUNKS = PER_W // B
PAIRS = (CHUNKS - 1) // 2
GROUPS = B // L

_HALF = 0.5
_THREEHALF = 1.5
_PI8 = 0.39269908169872414


def _rsqrt(u):
    yi = jnp.int32(0x5F3759DF) - (lax.bitcast_convert_type(u, jnp.int32) >> 1)
    y = lax.bitcast_convert_type(yi, jnp.float32)
    for _ in range(3):
        y = y * (_THREEHALF - _HALF * u * y * y)
    return y


def _log1p_exp_neg(ax):
    e = jnp.exp(-ax)
    s = e / (2.0 + e)
    s2 = s * s
    p = 1.0 / 13.0
    for c in (1.0 / 11.0, 1.0 / 9.0, 1.0 / 7.0, 1.0 / 5.0, 1.0 / 3.0, 1.0):
        p = c + s2 * p
    return 2.0 * s * p


def _sc_edges(pm_hbm, pv_hbm, het_hbm, idx0_hbm, idx1_hbm, links_hbm, out_hbm,
              idx0_v, idx1_v, links_v, pm0_v, pv0_v, pm1_v, pv1_v, het0_v,
              het1_v, out_v, sem_idx, sem_links, sem_rows, sem_out):
    wid = lax.axis_index("s") * NC + lax.axis_index("c")
    lanes = lax.iota(jnp.int32, L)

    def start_idx(i, s):
        base = wid * PER_W + i * B
        pltpu.make_async_copy(idx0_hbm.at[pl.ds(base, B)], idx0_v.at[s], sem_idx[s]).start()
        pltpu.make_async_copy(idx1_hbm.at[pl.ds(base, B)], idx1_v.at[s], sem_idx[s]).start()

    def wait_idx(s):
        pltpu.make_async_copy(idx0_hbm.at[pl.ds(0, B)], idx0_v.at[s], sem_idx[s]).wait()
        pltpu.make_async_copy(idx1_hbm.at[pl.ds(0, B)], idx1_v.at[s], sem_idx[s]).wait()

    def start_links(i, s):
        base = wid * PER_W + i * B
        pltpu.make_async_copy(links_hbm.at[pl.ds(base, B)], links_v.at[s], sem_links[s]).start()

    def wait_links(s):
        pltpu.make_async_copy(links_hbm.at[pl.ds(0, B)], links_v.at[s], sem_links[s]).wait()

    def row_copies(s):
        return (
            pltpu.make_async_copy(pm_hbm.at[idx0_v.at[s]], pm0_v.at[s], sem_rows[s]),
            pltpu.make_async_copy(pv_hbm.at[idx0_v.at[s]], pv0_v.at[s], sem_rows[s]),
            pltpu.make_async_copy(het_hbm.at[idx0_v.at[s]], het0_v.at[s], sem_rows[s]),
            pltpu.make_async_copy(pm_hbm.at[idx1_v.at[s]], pm1_v.at[s], sem_rows[s]),
            pltpu.make_async_copy(pv_hbm.at[idx1_v.at[s]], pv1_v.at[s], sem_rows[s]),
            pltpu.make_async_copy(het_hbm.at[idx1_v.at[s]], het1_v.at[s], sem_rows[s]),
        )

    def start_rows(s):
        for c in row_copies(s):
            c.start()

    def wait_rows(s):
        for c in row_copies(s):
            c.wait()

    def out_copy(i, s):
        base = wid * PER_W + i * B
        return pltpu.make_async_copy(out_v.at[s], out_hbm.at[pl.ds(base * 3, B * 3)],
                                     sem_out[s])

    def compute(s):
        zero = jnp.zeros((L,), jnp.float32)

        @plsc.parallel_loop(0, GROUPS, 1)
        def group_body(g):
            r = g * L + lanes
            acc = [zero] * 8
            for k in range(K):
                ck = jnp.full((L,), k, jnp.int32)
                a0 = plsc.load_gather(pm0_v.at[s], [r, ck])
                b0 = plsc.load_gather(pv0_v.at[s], [r, ck])
                a1 = plsc.load_gather(pm1_v.at[s], [r, ck])
                b1 = plsc.load_gather(pv1_v.at[s], [r, ck])
                p = 4 * (k & 1)
                acc[p] = acc[p] + a0 * a1
                acc[p + 1] = acc[p + 1] + a0 * a0 * b1
                acc[p + 2] = acc[p + 2] + a1 * a1 * b0
                acc[p + 3] = acc[p + 3] + b0 * b1
            m = acc[0] + acc[4]
            v = (acc[1] + acc[5]) + (acc[2] + acc[6]) + (acc[3] + acc[7])
            c0 = jnp.full((L,), 0, jnp.int32)
            c1 = jnp.full((L,), 1, jnp.int32)
            hm0 = plsc.load_gather(het0_v.at[s], [r, c0])
            hv0 = plsc.load_gather(het0_v.at[s], [r, c1])
            hm1 = plsc.load_gather(het1_v.at[s], [r, c0])
            hv1 = plsc.load_gather(het1_v.at[s], [r, c1])
            lm = m + hm0 + hm1
            lv = v + hv0 + hv1
            kap = _rsqrt(1.0 + _PI8 * lv)
            x = kap * lm
            sp = jnp.maximum(-x, 0.0) + _log1p_exp_neg(jnp.abs(x))
            y = links_v[s, pl.ds(g * L, L)].astype(jnp.float32)
            elbo = -sp - (1.0 - y) * x
            o = (g * L + lanes) * 3
            plsc.store_scatter(out_v.at[s], [o], lm)
            plsc.store_scatter(out_v.at[s], [o + 1], lv)
            plsc.store_scatter(out_v.at[s], [o + 2], elbo)

    def chunk_body(i, carry):
        start_idx(i, 0)
        wait_idx(0)
        start_links(i, 0)
        wait_links(0)
        start_rows(0)
        wait_rows(0)
        compute(0)
        c = out_copy(i, 0)
        c.start()
        c.wait()
        return carry

    lax.fori_loop(0, CHUNKS, chunk_body, 0)


_edges_kernel = functools.partial(
    pl.kernel,
    out_type=jax.ShapeDtypeStruct((E * 3,), jnp.float32),
    mesh=plsc.VectorSubcoreMesh(core_axis_name="c", subcore_axis_name="s"),
    scratch_types=[
        pltpu.VMEM((2, B), jnp.int32),
        pltpu.VMEM((2, B), jnp.int32),
        pltpu.VMEM((2, B), jnp.int32),
        pltpu.VMEM((2, B, K), jnp.float32),
        pltpu.VMEM((2, B, K), jnp.float32),
        pltpu.VMEM((2, B, K), jnp.float32),
        pltpu.VMEM((2, B, K), jnp.float32),
        pltpu.VMEM((2, B, 16), jnp.float32),
        pltpu.VMEM((2, B, 16), jnp.float32),
        pltpu.VMEM((2, B * 3), jnp.float32),
        [pltpu.SemaphoreType.DMA, pltpu.SemaphoreType.DMA],
        [pltpu.SemaphoreType.DMA, pltpu.SemaphoreType.DMA],
        [pltpu.SemaphoreType.DMA, pltpu.SemaphoreType.DMA],
        [pltpu.SemaphoreType.DMA, pltpu.SemaphoreType.DMA],
    ],
    compiler_params=pltpu.CompilerParams(needs_layout_passes=False,
                                         use_tc_tiling_on_sc=False),
)(_sc_edges)


def kernel(positions_mean, positions_var, heterogeneity_mean, heterogeneity_var,
           indices0, indices1, links):
    het = jnp.concatenate(
        [heterogeneity_mean.astype(jnp.float32),
         heterogeneity_var.astype(jnp.float32),
         jnp.zeros((N, 14), jnp.float32)], axis=1)
    idx0 = indices0.astype(jnp.int32)
    idx1 = indices1.astype(jnp.int32)
    flat = _edges_kernel(positions_mean, positions_var, het, idx0, idx1,
                         links.astype(jnp.int32))
    return flat.reshape(E, 3)


# speedup vs baseline: 1.2284x; 1.2284x over previous
import functools

import jax
import jax.numpy as jnp
from jax import lax
from jax.experimental import pallas as pl
from jax.experimental.pallas import tpu as pltpu
from jax.experimental.pallas import tpu_sc as plsc

N = 50000
K = 16
E = 1600000

NC = 2
NS = 16
NW = NC * NS
L = 16
B = 400

PER_W = E // NW
CHUNKS = PER_W // B
PAIRS = (CHUNKS - 1) // 2
GROUPS = B // L

_HALF = 0.5
_THREEHALF = 1.5
_PI8 = 0.39269908169872414


def _rsqrt(u):
    yi = jnp.int32(0x5F3759DF) - (lax.bitcast_convert_type(u, jnp.int32) >> 1)
    y = lax.bitcast_convert_type(yi, jnp.float32)
    for _ in range(3):
        y = y * (_THREEHALF - _HALF * u * y * y)
    return y


def _log1p_exp_neg(ax):
    e = jnp.exp(-ax)
    s = e / (2.0 + e)
    s2 = s * s
    p = 1.0 / 13.0
    for c in (1.0 / 11.0, 1.0 / 9.0, 1.0 / 7.0, 1.0 / 5.0, 1.0 / 3.0, 1.0):
        p = c + s2 * p
    return 2.0 * s * p


def _sc_edges(pm_hbm, pv_hbm, het_hbm, idx0_hbm, idx1_hbm, links_hbm, out_hbm,
              idx0_v, idx1_v, links_v, pm0_v, pv0_v, pm1_v, pv1_v, het0_v,
              het1_v, out_v, sem_idx, sem_links, sem_rows, sem_out):
    wid = lax.axis_index("s") * NC + lax.axis_index("c")
    lanes = lax.iota(jnp.int32, L)

    def start_idx(i, s):
        base = wid * PER_W + i * B
        pltpu.make_async_copy(idx0_hbm.at[pl.ds(base, B)], idx0_v.at[s], sem_idx[s]).start()
        pltpu.make_async_copy(idx1_hbm.at[pl.ds(base, B)], idx1_v.at[s], sem_idx[s]).start()

    def wait_idx(s):
        pltpu.make_async_copy(idx0_hbm.at[pl.ds(0, B)], idx0_v.at[s], sem_idx[s]).wait()
        pltpu.make_async_copy(idx1_hbm.at[pl.ds(0, B)], idx1_v.at[s], sem_idx[s]).wait()

    def start_links(i, s):
        base = wid * PER_W + i * B
        pltpu.make_async_copy(links_hbm.at[pl.ds(base, B)], links_v.at[s], sem_links[s]).start()

    def wait_links(s):
        pltpu.make_async_copy(links_hbm.at[pl.ds(0, B)], links_v.at[s], sem_links[s]).wait()

    def row_copies(s):
        return (
            pltpu.make_async_copy(pm_hbm.at[idx0_v.at[s]], pm0_v.at[s], sem_rows[s]),
            pltpu.make_async_copy(pv_hbm.at[idx0_v.at[s]], pv0_v.at[s], sem_rows[s]),
            pltpu.make_async_copy(het_hbm.at[idx0_v.at[s]], het0_v.at[s], sem_rows[s]),
            pltpu.make_async_copy(pm_hbm.at[idx1_v.at[s]], pm1_v.at[s], sem_rows[s]),
            pltpu.make_async_copy(pv_hbm.at[idx1_v.at[s]], pv1_v.at[s], sem_rows[s]),
            pltpu.make_async_copy(het_hbm.at[idx1_v.at[s]], het1_v.at[s], sem_rows[s]),
        )

    def start_rows(s):
        for c in row_copies(s):
            c.start()

    def wait_rows(s):
        for c in row_copies(s):
            c.wait()

    def out_copy(i, s):
        base = wid * PER_W + i * B
        return pltpu.make_async_copy(out_v.at[s], out_hbm.at[pl.ds(base * 3, B * 3)],
                                     sem_out[s])

    def compute(s):
        zero = jnp.zeros((L,), jnp.float32)

        @plsc.parallel_loop(0, GROUPS, 1)
        def group_body(g):
            r = g * L + lanes
            acc = [zero] * 8
            for k in range(K):
                ck = jnp.full((L,), k, jnp.int32)
                a0 = plsc.load_gather(pm0_v.at[s], [r, ck])
                b0 = plsc.load_gather(pv0_v.at[s], [r, ck])
                a1 = plsc.load_gather(pm1_v.at[s], [r, ck])
                b1 = plsc.load_gather(pv1_v.at[s], [r, ck])
                p = 4 * (k & 1)
                acc[p] = acc[p] + a0 * a1
                acc[p + 1] = acc[p + 1] + a0 * a0 * b1
                acc[p + 2] = acc[p + 2] + a1 * a1 * b0
                acc[p + 3] = acc[p + 3] + b0 * b1
            m = acc[0] + acc[4]
            v = (acc[1] + acc[5]) + (acc[2] + acc[6]) + (acc[3] + acc[7])
            c0 = jnp.full((L,), 0, jnp.int32)
            c1 = jnp.full((L,), 1, jnp.int32)
            hm0 = plsc.load_gather(het0_v.at[s], [r, c0])
            hv0 = plsc.load_gather(het0_v.at[s], [r, c1])
            hm1 = plsc.load_gather(het1_v.at[s], [r, c0])
            hv1 = plsc.load_gather(het1_v.at[s], [r, c1])
            lm = m + hm0 + hm1
            lv = v + hv0 + hv1
            kap = _rsqrt(1.0 + _PI8 * lv)
            x = kap * lm
            sp = jnp.maximum(-x, 0.0) + _log1p_exp_neg(jnp.abs(x))
            y = links_v[s, pl.ds(g * L, L)].astype(jnp.float32)
            elbo = -sp - (1.0 - y) * x
            o = (g * L + lanes) * 3
            plsc.store_scatter(out_v.at[s], [o], lm)
            plsc.store_scatter(out_v.at[s], [o + 1], lv)
            plsc.store_scatter(out_v.at[s], [o + 2], elbo)

    start_idx(0, 0)
    start_links(0, 0)
    start_idx(1, 1)
    start_links(1, 1)
    wait_idx(0)
    start_rows(0)

    def pair_body(j, carry):
        i = 2 * j
        wait_rows(0)
        wait_idx(1)
        start_rows(1)
        start_idx(i + 2, 0)

        @pl.when(j >= 1)
        def _():
            out_copy(i - 2, 0).wait()

        wait_links(0)
        compute(0)
        out_copy(i, 0).start()
        start_links(i + 2, 0)

        wait_rows(1)
        wait_idx(0)
        start_rows(0)

        @pl.when(j < PAIRS - 1)
        def _():
            start_idx(i + 3, 1)

        @pl.when(j >= 1)
        def _():
            out_copy(i - 1, 1).wait()

        wait_links(1)
        compute(1)
        out_copy(i + 1, 1).start()

        @pl.when(j < PAIRS - 1)
        def _():
            start_links(i + 3, 1)
        return carry

    lax.fori_loop(0, PAIRS, pair_body, 0)

    wait_rows(0)
    out_copy(CHUNKS - 3, 0).wait()
    wait_links(0)
    compute(0)
    out_copy(CHUNKS - 1, 0).start()
    out_copy(CHUNKS - 2, 1).wait()
    out_copy(CHUNKS - 1, 0).wait()


_edges_kernel = functools.partial(
    pl.kernel,
    out_type=jax.ShapeDtypeStruct((E * 3,), jnp.float32),
    mesh=plsc.VectorSubcoreMesh(core_axis_name="c", subcore_axis_name="s"),
    scratch_types=[
        pltpu.VMEM((2, B), jnp.int32),
        pltpu.VMEM((2, B), jnp.int32),
        pltpu.VMEM((2, B), jnp.int32),
        pltpu.VMEM((2, B, K), jnp.float32),
        pltpu.VMEM((2, B, K), jnp.float32),
        pltpu.VMEM((2, B, K), jnp.float32),
        pltpu.VMEM((2, B, K), jnp.float32),
        pltpu.VMEM((2, B, 16), jnp.float32),
        pltpu.VMEM((2, B, 16), jnp.float32),
        pltpu.VMEM((2, B * 3), jnp.float32),
        [pltpu.SemaphoreType.DMA, pltpu.SemaphoreType.DMA],
        [pltpu.SemaphoreType.DMA, pltpu.SemaphoreType.DMA],
        [pltpu.SemaphoreType.DMA, pltpu.SemaphoreType.DMA],
        [pltpu.SemaphoreType.DMA, pltpu.SemaphoreType.DMA],
    ],
    compiler_params=pltpu.CompilerParams(needs_layout_passes=False,
                                         use_tc_tiling_on_sc=False),
)(_sc_edges)


def kernel(positions_mean, positions_var, heterogeneity_mean, heterogeneity_var,
           indices0, indices1, links):
    het = jnp.concatenate(
        [heterogeneity_mean.astype(jnp.float32),
         heterogeneity_var.astype(jnp.float32),
         jnp.zeros((N, 14), jnp.float32)], axis=1)
    idx0 = indices0.astype(jnp.int32)
    idx1 = indices1.astype(jnp.int32)
    flat = _edges_kernel(positions_mean, positions_var, het, idx0, idx1,
                         links.astype(jnp.int32))
    return flat.reshape(E, 3)


# speedup vs baseline: 1.2465x; 1.0148x over previous
import functools

import jax
import jax.numpy as jnp
from jax import lax
from jax.experimental import pallas as pl
from jax.experimental.pallas import tpu as pltpu
from jax.experimental.pallas import tpu_sc as plsc

N = 50000
K = 16
E = 1600000

NC = 2
NS = 16
NW = NC * NS
L = 16
B = 400

PER_W = E // NW
CHUNKS = PER_W // B
PAIRS = (CHUNKS - 1) // 2
GROUPS = B // L

_HALF = 0.5
_THREEHALF = 1.5
_PI8 = 0.39269908169872414


def _rsqrt(u):
    yi = jnp.int32(0x5F3759DF) - (lax.bitcast_convert_type(u, jnp.int32) >> 1)
    y = lax.bitcast_convert_type(yi, jnp.float32)
    for _ in range(3):
        y = y * (_THREEHALF - _HALF * u * y * y)
    return y


def _log1p_exp_neg(ax):
    e = jnp.exp(-ax)
    s = e / (2.0 + e)
    s2 = s * s
    p = 1.0 / 13.0
    for c in (1.0 / 11.0, 1.0 / 9.0, 1.0 / 7.0, 1.0 / 5.0, 1.0 / 3.0, 1.0):
        p = c + s2 * p
    return 2.0 * s * p


def _sc_edges(pm_hbm, pv_hbm, hetw_hbm, idx0_hbm, idx1_hbm, links_hbm, out_hbm,
              idx0_v, idx1_v, links_v, pm0_v, pv0_v, pm1_v, pv1_v, het_tab,
              out_v, sem_idx, sem_links, sem_rows, sem_out, sem_het):
    wid = lax.axis_index("s") * NC + lax.axis_index("c")
    lanes = lax.iota(jnp.int32, L)

    def start_idx(i, s):
        base = wid * PER_W + i * B
        pltpu.make_async_copy(idx0_hbm.at[pl.ds(base, B)], idx0_v.at[s], sem_idx[s]).start()
        pltpu.make_async_copy(idx1_hbm.at[pl.ds(base, B)], idx1_v.at[s], sem_idx[s]).start()

    def wait_idx(s):
        pltpu.make_async_copy(idx0_hbm.at[pl.ds(0, B)], idx0_v.at[s], sem_idx[s]).wait()
        pltpu.make_async_copy(idx1_hbm.at[pl.ds(0, B)], idx1_v.at[s], sem_idx[s]).wait()

    def start_links(i, s):
        base = wid * PER_W + i * B
        pltpu.make_async_copy(links_hbm.at[pl.ds(base, B)], links_v.at[s], sem_links[s]).start()

    def wait_links(s):
        pltpu.make_async_copy(links_hbm.at[pl.ds(0, B)], links_v.at[s], sem_links[s]).wait()

    def row_copies(s):
        return (
            pltpu.make_async_copy(pm_hbm.at[idx0_v.at[s]], pm0_v.at[s], sem_rows[s]),
            pltpu.make_async_copy(pv_hbm.at[idx0_v.at[s]], pv0_v.at[s], sem_rows[s]),
            pltpu.make_async_copy(pm_hbm.at[idx1_v.at[s]], pm1_v.at[s], sem_rows[s]),
            pltpu.make_async_copy(pv_hbm.at[idx1_v.at[s]], pv1_v.at[s], sem_rows[s]),
        )

    def start_rows(s):
        for c in row_copies(s):
            c.start()

    def wait_rows(s):
        for c in row_copies(s):
            c.wait()

    def out_copy(i, s):
        base = wid * PER_W + i * B
        return pltpu.make_async_copy(out_v.at[s], out_hbm.at[pl.ds(base * 3, B * 3)],
                                     sem_out[s])

    def compute(s):
        zero = jnp.zeros((L,), jnp.float32)

        @plsc.parallel_loop(0, GROUPS, 1)
        def group_body(g):
            r = g * L + lanes
            acc = [zero] * 8
            for k in range(K):
                ck = jnp.full((L,), k, jnp.int32)
                a0 = plsc.load_gather(pm0_v.at[s], [r, ck])
                b0 = plsc.load_gather(pv0_v.at[s], [r, ck])
                a1 = plsc.load_gather(pm1_v.at[s], [r, ck])
                b1 = plsc.load_gather(pv1_v.at[s], [r, ck])
                p = 4 * (k & 1)
                acc[p] = acc[p] + a0 * a1
                acc[p + 1] = acc[p + 1] + a0 * a0 * b1
                acc[p + 2] = acc[p + 2] + a1 * a1 * b0
                acc[p + 3] = acc[p + 3] + b0 * b1
            m = acc[0] + acc[4]
            v = (acc[1] + acc[5]) + (acc[2] + acc[6]) + (acc[3] + acc[7])
            i0v = plsc.load_gather(idx0_v.at[s], [r])
            i1v = plsc.load_gather(idx1_v.at[s], [r])
            w0 = plsc.load_gather(het_tab, [i0v])
            w1 = plsc.load_gather(het_tab, [i1v])
            hi = jnp.full((L,), -65536, jnp.int32)
            s16 = jnp.full((L,), 16, jnp.int32)
            hm0 = lax.bitcast_convert_type(w0 & hi, jnp.float32)
            hv0 = lax.bitcast_convert_type(w0 << s16, jnp.float32)
            hm1 = lax.bitcast_convert_type(w1 & hi, jnp.float32)
            hv1 = lax.bitcast_convert_type(w1 << s16, jnp.float32)
            lm = m + hm0 + hm1
            lv = v + hv0 + hv1
            kap = _rsqrt(1.0 + _PI8 * lv)
            x = kap * lm
            sp = jnp.maximum(-x, 0.0) + _log1p_exp_neg(jnp.abs(x))
            y = links_v[s, pl.ds(g * L, L)].astype(jnp.float32)
            elbo = -sp - (1.0 - y) * x
            o = (g * L + lanes) * 3
            plsc.store_scatter(out_v.at[s], [o], lm)
            plsc.store_scatter(out_v.at[s], [o + 1], lv)
            plsc.store_scatter(out_v.at[s], [o + 2], elbo)

    pltpu.make_async_copy(hetw_hbm, het_tab, sem_het).start()

    start_idx(0, 0)
    start_links(0, 0)
    start_idx(1, 1)
    start_links(1, 1)
    wait_idx(0)
    start_rows(0)
    pltpu.make_async_copy(hetw_hbm, het_tab, sem_het).wait()

    def pair_body(j, carry):
        i = 2 * j
        wait_rows(0)
        wait_idx(1)
        start_rows(1)

        @pl.when(j >= 1)
        def _():
            out_copy(i - 2, 0).wait()

        wait_links(0)
        compute(0)
        out_copy(i, 0).start()
        start_idx(i + 2, 0)
        start_links(i + 2, 0)

        wait_rows(1)
        wait_idx(0)
        start_rows(0)

        @pl.when(j >= 1)
        def _():
            out_copy(i - 1, 1).wait()

        wait_links(1)
        compute(1)
        out_copy(i + 1, 1).start()

        @pl.when(j < PAIRS - 1)
        def _():
            start_idx(i + 3, 1)
            start_links(i + 3, 1)
        return carry

    lax.fori_loop(0, PAIRS, pair_body, 0)

    wait_rows(0)
    out_copy(CHUNKS - 3, 0).wait()
    wait_links(0)
    compute(0)
    out_copy(CHUNKS - 1, 0).start()
    out_copy(CHUNKS - 2, 1).wait()
    out_copy(CHUNKS - 1, 0).wait()


_edges_kernel = functools.partial(
    pl.kernel,
    out_type=jax.ShapeDtypeStruct((E * 3,), jnp.float32),
    mesh=plsc.VectorSubcoreMesh(core_axis_name="c", subcore_axis_name="s"),
    scratch_types=[
        pltpu.VMEM((2, B), jnp.int32),
        pltpu.VMEM((2, B), jnp.int32),
        pltpu.VMEM((2, B), jnp.int32),
        pltpu.VMEM((2, B, K), jnp.float32),
        pltpu.VMEM((2, B, K), jnp.float32),
        pltpu.VMEM((2, B, K), jnp.float32),
        pltpu.VMEM((2, B, K), jnp.float32),
        pltpu.VMEM((N,), jnp.int32),
        pltpu.VMEM((2, B * 3), jnp.float32),
        [pltpu.SemaphoreType.DMA, pltpu.SemaphoreType.DMA],
        [pltpu.SemaphoreType.DMA, pltpu.SemaphoreType.DMA],
        [pltpu.SemaphoreType.DMA, pltpu.SemaphoreType.DMA],
        [pltpu.SemaphoreType.DMA, pltpu.SemaphoreType.DMA],
        pltpu.SemaphoreType.DMA,
    ],
    compiler_params=pltpu.CompilerParams(needs_layout_passes=False,
                                         use_tc_tiling_on_sc=False),
)(_sc_edges)


def kernel(positions_mean, positions_var, heterogeneity_mean, heterogeneity_var,
           indices0, indices1, links):
    hm16 = lax.bitcast_convert_type(
        heterogeneity_mean.astype(jnp.bfloat16), jnp.uint16).astype(jnp.uint32)
    hv16 = lax.bitcast_convert_type(
        heterogeneity_var.astype(jnp.bfloat16), jnp.uint16).astype(jnp.uint32)
    hetw = lax.bitcast_convert_type((hm16 << 16) | hv16, jnp.int32).reshape(N)
    idx0 = indices0.astype(jnp.int32)
    idx1 = indices1.astype(jnp.int32)
    flat = _edges_kernel(positions_mean, positions_var, hetw, idx0, idx1,
                         links.astype(jnp.int32))
    return flat.reshape(E, 3)


# speedup vs baseline: 1.4085x; 1.1300x over previous
import functools

import jax
import jax.numpy as jnp
from jax import lax
from jax.experimental import pallas as pl
from jax.experimental.pallas import tpu as pltpu
from jax.experimental.pallas import tpu_sc as plsc

N = 50000
K = 16
E = 1600000

NC = 2
NS = 16
NW = NC * NS
L = 16
B = 400

PER_W = E // NW
CHUNKS = PER_W // B
PAIRS = (CHUNKS - 1) // 2
GROUPS = B // L

_HALF = 0.5
_THREEHALF = 1.5
_PI8 = 0.39269908169872414


def _rsqrt(u):
    yi = jnp.int32(0x5F3759DF) - (lax.bitcast_convert_type(u, jnp.int32) >> 1)
    y = lax.bitcast_convert_type(yi, jnp.float32)
    for _ in range(3):
        y = y * (_THREEHALF - _HALF * u * y * y)
    return y


def _log1p_exp_neg(ax):
    e = jnp.exp(-ax)
    s = e / (2.0 + e)
    s2 = s * s
    p = 1.0 / 13.0
    for c in (1.0 / 11.0, 1.0 / 9.0, 1.0 / 7.0, 1.0 / 5.0, 1.0 / 3.0, 1.0):
        p = c + s2 * p
    return 2.0 * s * p


def _sc_edges(pm_hbm, pv_hbm, hetw_hbm, idx0_hbm, idx1_hbm, links_hbm, out_hbm,
              idx0_v, idx1_v, links_v, pm0_v, pv0_v, pm1_v, pv1_v, het_tab,
              out_v, sem_idx, sem_links, sem_rows, sem_out, sem_het):
    wid = lax.axis_index("s") * NC + lax.axis_index("c")
    lanes = lax.iota(jnp.int32, L)

    def start_idx(i, s):
        base = wid * PER_W + i * B
        pltpu.make_async_copy(idx0_hbm.at[pl.ds(base, B)], idx0_v.at[s], sem_idx[s]).start()
        pltpu.make_async_copy(idx1_hbm.at[pl.ds(base, B)], idx1_v.at[s], sem_idx[s]).start()

    def wait_idx(s):
        pltpu.make_async_copy(idx0_hbm.at[pl.ds(0, B)], idx0_v.at[s], sem_idx[s]).wait()
        pltpu.make_async_copy(idx1_hbm.at[pl.ds(0, B)], idx1_v.at[s], sem_idx[s]).wait()

    def start_links(i, s):
        base = wid * PER_W + i * B
        pltpu.make_async_copy(links_hbm.at[pl.ds(base, B)], links_v.at[s], sem_links[s]).start()

    def wait_links(s):
        pltpu.make_async_copy(links_hbm.at[pl.ds(0, B)], links_v.at[s], sem_links[s]).wait()

    def row_copies(s):
        return (
            pltpu.make_async_copy(pm_hbm.at[idx0_v.at[s]], pm0_v.at[s], sem_rows[s]),
            pltpu.make_async_copy(pv_hbm.at[idx0_v.at[s]], pv0_v.at[s], sem_rows[s]),
            pltpu.make_async_copy(pm_hbm.at[idx1_v.at[s]], pm1_v.at[s], sem_rows[s]),
            pltpu.make_async_copy(pv_hbm.at[idx1_v.at[s]], pv1_v.at[s], sem_rows[s]),
        )

    def start_rows(s):
        for c in row_copies(s):
            c.start()

    def wait_rows(s):
        for c in row_copies(s):
            c.wait()

    def out_copy(i, s):
        base = wid * PER_W + i * B
        return pltpu.make_async_copy(out_v.at[s], out_hbm.at[pl.ds(base * 3, B * 3)],
                                     sem_out[s])

    def compute(s):
        zero = jnp.zeros((L,), jnp.float32)

        @plsc.parallel_loop(0, GROUPS, 1, unroll=5)
        def group_body(g):
            r = g * L + lanes
            acc = [zero] * 8
            for k in range(K):
                ck = jnp.full((L,), k, jnp.int32)
                a0 = plsc.load_gather(pm0_v.at[s], [r, ck])
                b0 = plsc.load_gather(pv0_v.at[s], [r, ck])
                a1 = plsc.load_gather(pm1_v.at[s], [r, ck])
                b1 = plsc.load_gather(pv1_v.at[s], [r, ck])
                p = 4 * (k & 1)
                acc[p] = acc[p] + a0 * a1
                acc[p + 1] = acc[p + 1] + a0 * a0 * b1
                acc[p + 2] = acc[p + 2] + a1 * a1 * b0
                acc[p + 3] = acc[p + 3] + b0 * b1
            m = acc[0] + acc[4]
            v = (acc[1] + acc[5]) + (acc[2] + acc[6]) + (acc[3] + acc[7])
            i0v = plsc.load_gather(idx0_v.at[s], [r])
            i1v = plsc.load_gather(idx1_v.at[s], [r])
            w0 = plsc.load_gather(het_tab, [i0v])
            w1 = plsc.load_gather(het_tab, [i1v])
            hi = jnp.full((L,), -65536, jnp.int32)
            s16 = jnp.full((L,), 16, jnp.int32)
            hm0 = lax.bitcast_convert_type(w0 & hi, jnp.float32)
            hv0 = lax.bitcast_convert_type(w0 << s16, jnp.float32)
            hm1 = lax.bitcast_convert_type(w1 & hi, jnp.float32)
            hv1 = lax.bitcast_convert_type(w1 << s16, jnp.float32)
            lm = m + hm0 + hm1
            lv = v + hv0 + hv1
            kap = _rsqrt(1.0 + _PI8 * lv)
            x = kap * lm
            sp = jnp.maximum(-x, 0.0) + _log1p_exp_neg(jnp.abs(x))
            y = links_v[s, pl.ds(g * L, L)].astype(jnp.float32)
            elbo = -sp - (1.0 - y) * x
            o = (g * L + lanes) * 3
            plsc.store_scatter(out_v.at[s], [o], lm)
            plsc.store_scatter(out_v.at[s], [o + 1], lv)
            plsc.store_scatter(out_v.at[s], [o + 2], elbo)

    pltpu.make_async_copy(hetw_hbm, het_tab, sem_het).start()

    start_idx(0, 0)
    start_links(0, 0)
    start_idx(1, 1)
    start_links(1, 1)
    wait_idx(0)
    start_rows(0)
    pltpu.make_async_copy(hetw_hbm, het_tab, sem_het).wait()

    def pair_body(j, carry):
        i = 2 * j
        wait_rows(0)
        wait_idx(1)
        start_rows(1)

        @pl.when(j >= 1)
        def _():
            out_copy(i - 2, 0).wait()

        wait_links(0)
        compute(0)
        out_copy(i, 0).start()
        start_idx(i + 2, 0)
        start_links(i + 2, 0)

        wait_rows(1)
        wait_idx(0)
        start_rows(0)

        @pl.when(j >= 1)
        def _():
            out_copy(i - 1, 1).wait()

        wait_links(1)
        compute(1)
        out_copy(i + 1, 1).start()

        @pl.when(j < PAIRS - 1)
        def _():
            start_idx(i + 3, 1)
            start_links(i + 3, 1)
        return carry

    lax.fori_loop(0, PAIRS, pair_body, 0)

    wait_rows(0)
    out_copy(CHUNKS - 3, 0).wait()
    wait_links(0)
    compute(0)
    out_copy(CHUNKS - 1, 0).start()
    out_copy(CHUNKS - 2, 1).wait()
    out_copy(CHUNKS - 1, 0).wait()


_edges_kernel = functools.partial(
    pl.kernel,
    out_type=jax.ShapeDtypeStruct((E * 3,), jnp.float32),
    mesh=plsc.VectorSubcoreMesh(core_axis_name="c", subcore_axis_name="s"),
    scratch_types=[
        pltpu.VMEM((2, B), jnp.int32),
        pltpu.VMEM((2, B), jnp.int32),
        pltpu.VMEM((2, B), jnp.int32),
        pltpu.VMEM((2, B, K), jnp.float32),
        pltpu.VMEM((2, B, K), jnp.float32),
        pltpu.VMEM((2, B, K), jnp.float32),
        pltpu.VMEM((2, B, K), jnp.float32),
        pltpu.VMEM((N,), jnp.int32),
        pltpu.VMEM((2, B * 3), jnp.float32),
        [pltpu.SemaphoreType.DMA, pltpu.SemaphoreType.DMA],
        [pltpu.SemaphoreType.DMA, pltpu.SemaphoreType.DMA],
        [pltpu.SemaphoreType.DMA, pltpu.SemaphoreType.DMA],
        [pltpu.SemaphoreType.DMA, pltpu.SemaphoreType.DMA],
        pltpu.SemaphoreType.DMA,
    ],
    compiler_params=pltpu.CompilerParams(needs_layout_passes=False,
                                         use_tc_tiling_on_sc=False),
)(_sc_edges)


def kernel(positions_mean, positions_var, heterogeneity_mean, heterogeneity_var,
           indices0, indices1, links):
    hm16 = lax.bitcast_convert_type(
        heterogeneity_mean.astype(jnp.bfloat16), jnp.uint16).astype(jnp.uint32)
    hv16 = lax.bitcast_convert_type(
        heterogeneity_var.astype(jnp.bfloat16), jnp.uint16).astype(jnp.uint32)
    hetw = lax.bitcast_convert_type((hm16 << 16) | hv16, jnp.int32).reshape(N)
    idx0 = indices0.astype(jnp.int32)
    idx1 = indices1.astype(jnp.int32)
    flat = _edges_kernel(positions_mean, positions_var, hetw, idx0, idx1,
                         links.astype(jnp.int32))
    return flat.reshape(E, 3)
